# baseline (speedup 1.0000x reference)
import functools

import jax
import jax.numpy as jnp
from jax import lax
from jax.experimental import pallas as pl
from jax.experimental.pallas import tpu as pltpu
from jax.experimental.pallas import tpu_sc as plsc

N1 = 2048
N2 = 8192
IN_F = 512
OUT_F = 256
K = 3

QBLK = 512
_NSTEPS = N2 // QBLK


def _y2_body(x2_ref, w2_ref, b2_ref, g2_ref, be2_ref, y2_ref, rsgb_ref):
    y2 = jnp.dot(x2_ref[...], w2_ref[...],
                 preferred_element_type=jnp.float32) + b2_ref[...]
    y2_ref[...] = y2
    mu = jnp.sum(y2, 0, keepdims=True) * (1.0 / N2)
    d = y2 - mu
    var = jnp.sum(d * d, 0, keepdims=True) * (1.0 / N2)
    rsg = g2_ref[...] / jnp.sqrt(var + 1e-5)
    rsgb_ref[0:1, :] = rsg
    rsgb_ref[1:2, :] = be2_ref[...] - mu * rsg


def _y2_stage(x2, W2, b2, g2, be2):
    r = lambda v: v.reshape(1, OUT_F)
    return pl.pallas_call(
        _y2_body,
        out_shape=[
            jax.ShapeDtypeStruct((N2, OUT_F), jnp.float32),
            jax.ShapeDtypeStruct((2, OUT_F), jnp.float32),
        ],
    )(x2, W2, r(b2), r(g2), r(be2))


def _tc_body(x1_ref, w1_ref, b1_ref, g1_ref, be1_ref,
             p2_ref, p1t_ref,
             f1_ref, idx_ref, wn_ref):
    i = pl.program_id(0)

    @pl.when(i == 0)
    def _():
        y = jnp.dot(x1_ref[...], w1_ref[...],
                    preferred_element_type=jnp.float32) + b1_ref[...]
        mu = jnp.sum(y, 0, keepdims=True) * (1.0 / N1)
        d = y - mu
        var = jnp.sum(d * d, 0, keepdims=True) * (1.0 / N1)
        yn = d / jnp.sqrt(var + 1e-5)
        f1_ref[...] = jnp.maximum(g1_ref[...] * yn + be1_ref[...], 0.0)

    qx = p2_ref[:, 0:1]
    qy = p2_ref[:, 1:2]
    qz = p2_ref[:, 2:3]
    dx = qx - p1t_ref[0:1, :]
    dy = qy - p1t_ref[1:2, :]
    dz = qz - p1t_ref[2:3, :]
    d2 = dx * dx + dy * dy + dz * dz
    lane = lax.broadcasted_iota(jnp.int32, (QBLK, N1), 1)
    ws = []
    for j in range(K):
        m = jnp.min(d2, axis=1, keepdims=True)
        cand = jnp.where(d2 == m, lane, N1)
        imin = jnp.min(cand, axis=1, keepdims=True)
        if j < K - 1:
            d2 = jnp.where(cand == imin, jnp.inf, d2)
        w = 1.0 / jnp.maximum(m, 1e-16)
        idx_ref[:, j : j + 1] = imin
        ws.append(w)
    den = (ws[0] + ws[1]) + ws[2]
    for j in range(K):
        wn_ref[:, j, :] = jnp.broadcast_to(ws[j] / den, (QBLK, 16))


def _tc_fused(x1, W1, b1, g1, be1, p1, p2):
    p1t = p1.T
    r = lambda v: v.reshape(1, OUT_F)
    return pl.pallas_call(
        _tc_body,
        grid=(_NSTEPS,),
        in_specs=[
            pl.BlockSpec((N1, IN_F), lambda i: (0, 0)),
            pl.BlockSpec((IN_F, OUT_F), lambda i: (0, 0)),
            pl.BlockSpec((1, OUT_F), lambda i: (0, 0)),
            pl.BlockSpec((1, OUT_F), lambda i: (0, 0)),
            pl.BlockSpec((1, OUT_F), lambda i: (0, 0)),
            pl.BlockSpec((QBLK, 3), lambda i: (i, 0)),
            pl.BlockSpec((3, N1), lambda i: (0, 0)),
        ],
        out_specs=[
            pl.BlockSpec((N1, OUT_F), lambda i: (0, 0)),
            pl.BlockSpec((QBLK, K), lambda i: (i, 0)),
            pl.BlockSpec((QBLK, K, 16), lambda i: (i, 0, 0)),
        ],
        out_shape=[
            jax.ShapeDtypeStruct((N1, OUT_F), jnp.float32),
            jax.ShapeDtypeStruct((N2, K), jnp.int32),
            jax.ShapeDtypeStruct((N2, K, 16), jnp.float32),
        ],
    )(x1, W1, r(b1), r(g1), r(be1), p2, p1t)


_NC = 2
_NS = 16
_NW = _NC * _NS
_L = 16
_QW = N2 // _NW
_CH = 32
_NCH = _QW // _CH


def _sc_body(f1_hbm, idxf_hbm, wnw_hbm, y2_hbm, rsgb_hbm, out_hbm,
             idx_v0, idx_v1, wn_v0, wn_v1, rows_v0, rows_v1, y2_v0, y2_v1,
             rsgb_v, gsem0, gsem1, wsem0, wsem1):
    wid = lax.axis_index("s") * _NC + lax.axis_index("c")
    base_q = wid * _QW
    pltpu.sync_copy(rsgb_hbm, rsgb_v)
    rsgs = [rsgb_v[0, pl.ds(c * _L, _L)] for c in range(OUT_F // _L)]
    bps = [rsgb_v[1, pl.ds(c * _L, _L)] for c in range(OUT_F // _L)]
    idx_b = [idx_v0, idx_v1]
    wn_b = [wn_v0, wn_v1]
    rows_b = [rows_v0, rows_v1]
    y2_b = [y2_v0, y2_v1]
    gsems = [gsem0, gsem1]
    wsems = [wsem0, wsem1]

    def stage(ci):
        par = ci % 2
        q0 = base_q + ci * _CH
        pltpu.sync_copy(idxf_hbm.at[pl.ds(q0 * K, _CH * K)], idx_b[par])
        g = pltpu.async_copy(f1_hbm.at[idx_b[par]], rows_b[par], gsems[par])
        pltpu.sync_copy(wnw_hbm.at[pl.ds(q0, _CH)], wn_b[par])
        pltpu.sync_copy(y2_hbm.at[pl.ds(q0, _CH)], y2_b[par])
        return g

    g = [None, None]
    wb = [None, None]
    g[0] = stage(0)
    for ci in range(_NCH):
        par = ci % 2
        g[par].wait()
        if ci + 1 < _NCH:
            if wb[(ci + 1) % 2] is not None:
                wb[(ci + 1) % 2].wait()
            g[(ci + 1) % 2] = stage(ci + 1)
        wnb, rowsb, y2b = wn_b[par], rows_b[par], y2_b[par]

        @plsc.parallel_loop(0, _CH, unroll=2)
        def _(qi, wnb=wnb, rowsb=rowsb, y2b=y2b):
            i3 = qi * K
            w0 = wnb[qi, 0, :]
            w1 = wnb[qi, 1, :]
            w2 = wnb[qi, 2, :]
            for c in range(OUT_F // _L):
                sl = pl.ds(c * _L, _L)
                f2 = jnp.maximum(y2b[qi, sl] * rsgs[c] + bps[c], 0.0)
                acc = f2 + w0 * rowsb[i3, sl]
                acc = acc + w1 * rowsb[i3 + 1, sl]
                acc = acc + w2 * rowsb[i3 + 2, sl]
                y2b[qi, sl] = acc
        q0 = base_q + ci * _CH
        wb[par] = pltpu.async_copy(y2b, out_hbm.at[pl.ds(q0, _CH)], wsems[par])
    wb[0].wait()
    wb[1].wait()


def _sc_gather(f1, idxf, wnw, y2, rsgb):
    mesh = plsc.VectorSubcoreMesh(core_axis_name="c", subcore_axis_name="s")
    fn = pl.kernel(
        _sc_body,
        out_type=jax.ShapeDtypeStruct((N2, OUT_F), jnp.float32),
        mesh=mesh,
        scratch_types=[
            pltpu.VMEM((_CH * K,), jnp.int32),
            pltpu.VMEM((_CH * K,), jnp.int32),
            pltpu.VMEM((_CH, K, _L), jnp.float32),
            pltpu.VMEM((_CH, K, _L), jnp.float32),
            pltpu.VMEM((_CH * K, OUT_F), jnp.float32),
            pltpu.VMEM((_CH * K, OUT_F), jnp.float32),
            pltpu.VMEM((_CH, OUT_F), jnp.float32),
            pltpu.VMEM((_CH, OUT_F), jnp.float32),
            pltpu.VMEM((2, OUT_F), jnp.float32),
            pltpu.SemaphoreType.DMA,
            pltpu.SemaphoreType.DMA,
            pltpu.SemaphoreType.DMA,
            pltpu.SemaphoreType.DMA,
        ],
    )
    return fn(f1, idxf, wnw, y2, rsgb)


def kernel(features_1, positions_1, batch_1, features_2, positions_2, batch_2,
           W1, b1, g1, be1, W2, b2, g2, be2):
    f1, idx, wnw = _tc_fused(features_1, W1, b1, g1, be1,
                             positions_1, positions_2)
    y2, rsgb = _y2_stage(features_2, W2, b2, g2, be2)
    out = _sc_gather(f1, idx.reshape(-1), wnw, y2, rsgb)
    return (out, positions_2, batch_2)


# speedup vs baseline: 1.0637x; 1.0637x over previous
import functools

import jax
import jax.numpy as jnp
from jax import lax
from jax.experimental import pallas as pl
from jax.experimental.pallas import tpu as pltpu
from jax.experimental.pallas import tpu_sc as plsc

N1 = 2048
N2 = 8192
IN_F = 512
OUT_F = 256
K = 3

QBLK = 512
_NSTEPS = N2 // QBLK


def _tc_body(x1_ref, w1_ref, b1_ref, g1_ref, be1_ref,
             x2_ref, w2_ref, b2_ref, g2_ref, be2_ref,
             p2_ref, p1t_ref,
             f1_ref, idx_ref, wn_ref, y2_ref, rsgb_ref,
             s1_ref, s2_ref):
    i = pl.program_id(0)

    @pl.when(i == 0)
    def _():
        y = jnp.dot(x1_ref[...], w1_ref[...],
                    preferred_element_type=jnp.float32) + b1_ref[...]
        mu = jnp.sum(y, 0, keepdims=True) * (1.0 / N1)
        d = y - mu
        var = jnp.sum(d * d, 0, keepdims=True) * (1.0 / N1)
        yn = d / jnp.sqrt(var + 1e-5)
        f1_ref[...] = jnp.maximum(g1_ref[...] * yn + be1_ref[...], 0.0)

    y2 = jnp.dot(x2_ref[...], w2_ref[...],
                 preferred_element_type=jnp.float32) + b2_ref[...]
    y2_ref[...] = y2
    ps1 = jnp.sum(y2, 0, keepdims=True)
    ps2 = jnp.sum(y2 * y2, 0, keepdims=True)

    @pl.when(i == 0)
    def _():
        s1_ref[...] = ps1
        s2_ref[...] = ps2

    @pl.when(i > 0)
    def _():
        s1_ref[...] += ps1
        s2_ref[...] += ps2

    @pl.when(i == _NSTEPS - 1)
    def _():
        mu = s1_ref[...] * (1.0 / N2)
        var = s2_ref[...] * (1.0 / N2) - mu * mu
        rsg = g2_ref[...] / jnp.sqrt(var + 1e-5)
        rsgb_ref[0:1, :] = rsg
        rsgb_ref[1:2, :] = be2_ref[...] - mu * rsg

    qx = p2_ref[:, 0:1]
    qy = p2_ref[:, 1:2]
    qz = p2_ref[:, 2:3]
    dx = qx - p1t_ref[0:1, :]
    dy = qy - p1t_ref[1:2, :]
    dz = qz - p1t_ref[2:3, :]
    d2 = dx * dx + dy * dy + dz * dz
    lane = lax.broadcasted_iota(jnp.int32, (QBLK, N1), 1)
    ws = []
    for j in range(K):
        m = jnp.min(d2, axis=1, keepdims=True)
        cand = jnp.where(d2 == m, lane, N1)
        imin = jnp.min(cand, axis=1, keepdims=True)
        if j < K - 1:
            d2 = jnp.where(cand == imin, jnp.inf, d2)
        w = 1.0 / jnp.maximum(m, 1e-16)
        idx_ref[:, j : j + 1] = imin
        ws.append(w)
    den = (ws[0] + ws[1]) + ws[2]
    for j in range(K):
        wn_ref[:, j, :] = jnp.broadcast_to(ws[j] / den, (QBLK, 16))


def _tc_fused(x1, W1, b1, g1, be1, x2, W2, b2, g2, be2, p1, p2):
    p1t = p1.T
    r = lambda v: v.reshape(1, OUT_F)
    return pl.pallas_call(
        _tc_body,
        grid=(_NSTEPS,),
        in_specs=[
            pl.BlockSpec((N1, IN_F), lambda i: (0, 0)),
            pl.BlockSpec((IN_F, OUT_F), lambda i: (0, 0)),
            pl.BlockSpec((1, OUT_F), lambda i: (0, 0)),
            pl.BlockSpec((1, OUT_F), lambda i: (0, 0)),
            pl.BlockSpec((1, OUT_F), lambda i: (0, 0)),
            pl.BlockSpec((QBLK, IN_F), lambda i: (i, 0)),
            pl.BlockSpec((IN_F, OUT_F), lambda i: (0, 0)),
            pl.BlockSpec((1, OUT_F), lambda i: (0, 0)),
            pl.BlockSpec((1, OUT_F), lambda i: (0, 0)),
            pl.BlockSpec((1, OUT_F), lambda i: (0, 0)),
            pl.BlockSpec((QBLK, 3), lambda i: (i, 0)),
            pl.BlockSpec((3, N1), lambda i: (0, 0)),
        ],
        out_specs=[
            pl.BlockSpec((N1, OUT_F), lambda i: (0, 0)),
            pl.BlockSpec((QBLK, K), lambda i: (i, 0)),
            pl.BlockSpec((QBLK, K, 16), lambda i: (i, 0, 0)),
            pl.BlockSpec((QBLK, OUT_F), lambda i: (i, 0)),
            pl.BlockSpec((2, OUT_F), lambda i: (0, 0)),
        ],
        out_shape=[
            jax.ShapeDtypeStruct((N1, OUT_F), jnp.float32),
            jax.ShapeDtypeStruct((N2, K), jnp.int32),
            jax.ShapeDtypeStruct((N2, K, 16), jnp.float32),
            jax.ShapeDtypeStruct((N2, OUT_F), jnp.float32),
            jax.ShapeDtypeStruct((2, OUT_F), jnp.float32),
        ],
        scratch_shapes=[
            pltpu.VMEM((1, OUT_F), jnp.float32),
            pltpu.VMEM((1, OUT_F), jnp.float32),
        ],
    )(x1, W1, r(b1), r(g1), r(be1), x2, W2, r(b2), r(g2), r(be2), p2, p1t)


_NC = 2
_NS = 16
_NW = _NC * _NS
_L = 16
_QW = N2 // _NW
_CH = 32
_NCH = _QW // _CH


def _sc_body(f1_hbm, idxf_hbm, wnw_hbm, y2_hbm, rsgb_hbm, out_hbm,
             idx_v0, idx_v1, wn_v0, wn_v1, rows_v0, rows_v1, y2_v0, y2_v1,
             rsgb_v, gsem0, gsem1, wsem0, wsem1):
    wid = lax.axis_index("s") * _NC + lax.axis_index("c")
    base_q = wid * _QW
    pltpu.sync_copy(rsgb_hbm, rsgb_v)
    rsgs = [rsgb_v[0, pl.ds(c * _L, _L)] for c in range(OUT_F // _L)]
    bps = [rsgb_v[1, pl.ds(c * _L, _L)] for c in range(OUT_F // _L)]
    idx_b = [idx_v0, idx_v1]
    wn_b = [wn_v0, wn_v1]
    rows_b = [rows_v0, rows_v1]
    y2_b = [y2_v0, y2_v1]
    gsems = [gsem0, gsem1]
    wsems = [wsem0, wsem1]

    def stage(ci):
        par = ci % 2
        q0 = base_q + ci * _CH
        pltpu.sync_copy(idxf_hbm.at[pl.ds(q0 * K, _CH * K)], idx_b[par])
        g = pltpu.async_copy(f1_hbm.at[idx_b[par]], rows_b[par], gsems[par])
        pltpu.sync_copy(wnw_hbm.at[pl.ds(q0, _CH)], wn_b[par])
        pltpu.sync_copy(y2_hbm.at[pl.ds(q0, _CH)], y2_b[par])
        return g

    g = [None, None]
    wb = [None, None]
    g[0] = stage(0)
    for ci in range(_NCH):
        par = ci % 2
        g[par].wait()
        if ci + 1 < _NCH:
            if wb[(ci + 1) % 2] is not None:
                wb[(ci + 1) % 2].wait()
            g[(ci + 1) % 2] = stage(ci + 1)
        wnb, rowsb, y2b = wn_b[par], rows_b[par], y2_b[par]

        @plsc.parallel_loop(0, _CH, unroll=4)
        def _(qi, wnb=wnb, rowsb=rowsb, y2b=y2b):
            i3 = qi * K
            w0 = wnb[qi, 0, :]
            w1 = wnb[qi, 1, :]
            w2 = wnb[qi, 2, :]
            for c in range(OUT_F // _L):
                sl = pl.ds(c * _L, _L)
                f2 = jnp.maximum(y2b[qi, sl] * rsgs[c] + bps[c], 0.0)
                acc = f2 + w0 * rowsb[i3, sl]
                acc = acc + w1 * rowsb[i3 + 1, sl]
                acc = acc + w2 * rowsb[i3 + 2, sl]
                y2b[qi, sl] = acc
        q0 = base_q + ci * _CH
        wb[par] = pltpu.async_copy(y2b, out_hbm.at[pl.ds(q0, _CH)], wsems[par])
    wb[0].wait()
    wb[1].wait()


def _sc_gather(f1, idxf, wnw, y2, rsgb):
    mesh = plsc.VectorSubcoreMesh(core_axis_name="c", subcore_axis_name="s")
    fn = pl.kernel(
        _sc_body,
        out_type=jax.ShapeDtypeStruct((N2, OUT_F), jnp.float32),
        mesh=mesh,
        scratch_types=[
            pltpu.VMEM((_CH * K,), jnp.int32),
            pltpu.VMEM((_CH * K,), jnp.int32),
            pltpu.VMEM((_CH, K, _L), jnp.float32),
            pltpu.VMEM((_CH, K, _L), jnp.float32),
            pltpu.VMEM((_CH * K, OUT_F), jnp.float32),
            pltpu.VMEM((_CH * K, OUT_F), jnp.float32),
            pltpu.VMEM((_CH, OUT_F), jnp.float32),
            pltpu.VMEM((_CH, OUT_F), jnp.float32),
            pltpu.VMEM((2, OUT_F), jnp.float32),
            pltpu.SemaphoreType.DMA,
            pltpu.SemaphoreType.DMA,
            pltpu.SemaphoreType.DMA,
            pltpu.SemaphoreType.DMA,
        ],
    )
    return fn(f1, idxf, wnw, y2, rsgb)


def kernel(features_1, positions_1, batch_1, features_2, positions_2, batch_2,
           W1, b1, g1, be1, W2, b2, g2, be2):
    f1, idx, wnw, y2, rsgb = _tc_fused(
        features_1, W1, b1, g1, be1,
        features_2, W2, b2, g2, be2,
        positions_1, positions_2)
    out = _sc_gather(f1, idx.reshape(-1), wnw, y2, rsgb)
    return (out, positions_2, batch_2)


# speedup vs baseline: 1.1807x; 1.1100x over previous
import functools

import jax
import jax.numpy as jnp
from jax import lax
from jax.experimental import pallas as pl
from jax.experimental.pallas import tpu as pltpu
from jax.experimental.pallas import tpu_sc as plsc

N1 = 2048
N2 = 8192
IN_F = 512
OUT_F = 256
K = 3

QBLK = 1024
_NSTEPS = N2 // QBLK


def _tc_body(x1_ref, w1_ref, b1_ref, g1_ref, be1_ref,
             x2_ref, w2_ref, b2_ref, g2_ref, be2_ref,
             p2_ref, p1t_ref,
             f1_ref, idx_ref, wn_ref, y2_ref, rsgb_ref,
             s1_ref, s2_ref):
    i = pl.program_id(0)

    @pl.when(i == 0)
    def _():
        y = jnp.dot(x1_ref[...], w1_ref[...],
                    preferred_element_type=jnp.float32) + b1_ref[...]
        mu = jnp.sum(y, 0, keepdims=True) * (1.0 / N1)
        d = y - mu
        var = jnp.sum(d * d, 0, keepdims=True) * (1.0 / N1)
        yn = d / jnp.sqrt(var + 1e-5)
        f1_ref[...] = jnp.maximum(g1_ref[...] * yn + be1_ref[...], 0.0)

    y2 = jnp.dot(x2_ref[...], w2_ref[...],
                 preferred_element_type=jnp.float32) + b2_ref[...]
    y2_ref[...] = y2
    ps1 = jnp.sum(y2, 0, keepdims=True)
    ps2 = jnp.sum(y2 * y2, 0, keepdims=True)

    @pl.when(i == 0)
    def _():
        s1_ref[...] = ps1
        s2_ref[...] = ps2

    @pl.when(i > 0)
    def _():
        s1_ref[...] += ps1
        s2_ref[...] += ps2

    @pl.when(i == _NSTEPS - 1)
    def _():
        mu = s1_ref[...] * (1.0 / N2)
        var = s2_ref[...] * (1.0 / N2) - mu * mu
        rsg = g2_ref[...] / jnp.sqrt(var + 1e-5)
        rsgb_ref[0:1, :] = rsg
        rsgb_ref[1:2, :] = be2_ref[...] - mu * rsg

    qx = p2_ref[:, 0:1]
    qy = p2_ref[:, 1:2]
    qz = p2_ref[:, 2:3]
    dx = qx - p1t_ref[0:1, :]
    dy = qy - p1t_ref[1:2, :]
    dz = qz - p1t_ref[2:3, :]
    d2 = dx * dx + dy * dy + dz * dz
    lane = lax.broadcasted_iota(jnp.int32, (QBLK, N1), 1)
    ws = []
    for j in range(K):
        m = jnp.min(d2, axis=1, keepdims=True)
        cand = jnp.where(d2 == m, lane, N1)
        imin = jnp.min(cand, axis=1, keepdims=True)
        if j < K - 1:
            d2 = jnp.where(cand == imin, jnp.inf, d2)
        w = 1.0 / jnp.maximum(m, 1e-16)
        idx_ref[:, j : j + 1] = imin
        ws.append(w)
    den = (ws[0] + ws[1]) + ws[2]
    for j in range(K):
        wn_ref[:, j : j + 1] = ws[j] / den


def _tc_fused(x1, W1, b1, g1, be1, x2, W2, b2, g2, be2, p1, p2):
    p1t = p1.T
    r = lambda v: v.reshape(1, OUT_F)
    return pl.pallas_call(
        _tc_body,
        grid=(_NSTEPS,),
        in_specs=[
            pl.BlockSpec((N1, IN_F), lambda i: (0, 0)),
            pl.BlockSpec((IN_F, OUT_F), lambda i: (0, 0)),
            pl.BlockSpec((1, OUT_F), lambda i: (0, 0)),
            pl.BlockSpec((1, OUT_F), lambda i: (0, 0)),
            pl.BlockSpec((1, OUT_F), lambda i: (0, 0)),
            pl.BlockSpec((QBLK, IN_F), lambda i: (i, 0)),
            pl.BlockSpec((IN_F, OUT_F), lambda i: (0, 0)),
            pl.BlockSpec((1, OUT_F), lambda i: (0, 0)),
            pl.BlockSpec((1, OUT_F), lambda i: (0, 0)),
            pl.BlockSpec((1, OUT_F), lambda i: (0, 0)),
            pl.BlockSpec((QBLK, 3), lambda i: (i, 0)),
            pl.BlockSpec((3, N1), lambda i: (0, 0)),
        ],
        out_specs=[
            pl.BlockSpec((N1, OUT_F), lambda i: (0, 0)),
            pl.BlockSpec((QBLK, K), lambda i: (i, 0)),
            pl.BlockSpec((QBLK, K), lambda i: (i, 0)),
            pl.BlockSpec((QBLK, OUT_F), lambda i: (i, 0)),
            pl.BlockSpec((2, OUT_F), lambda i: (0, 0)),
        ],
        out_shape=[
            jax.ShapeDtypeStruct((N1, OUT_F), jnp.float32),
            jax.ShapeDtypeStruct((N2, K), jnp.int32),
            jax.ShapeDtypeStruct((N2, K), jnp.float32),
            jax.ShapeDtypeStruct((N2, OUT_F), jnp.float32),
            jax.ShapeDtypeStruct((2, OUT_F), jnp.float32),
        ],
        scratch_shapes=[
            pltpu.VMEM((1, OUT_F), jnp.float32),
            pltpu.VMEM((1, OUT_F), jnp.float32),
        ],
    )(x1, W1, r(b1), r(g1), r(be1), x2, W2, r(b2), r(g2), r(be2), p2, p1t)


_NC = 2
_NS = 16
_NW = _NC * _NS
_L = 16
_QW = N2 // _NW
_CH = 32
_NCH = _QW // _CH


def _sc_body(f1_hbm, idxf_hbm, wnf_hbm, y2_hbm, rsgb_hbm, out_hbm,
             idx_v0, idx_v1, wn_v0, wn_v1, rows_v0, rows_v1, y2_v0, y2_v1,
             rsgb_v, gsem0, gsem1, wsem0, wsem1):
    wid = lax.axis_index("s") * _NC + lax.axis_index("c")
    base_q = wid * _QW
    splat0 = jnp.zeros((_L,), jnp.int32)
    splat1 = jnp.ones((_L,), jnp.int32)
    splat2 = jnp.full((_L,), 2, jnp.int32)
    pltpu.sync_copy(rsgb_hbm, rsgb_v)
    rsgs = [rsgb_v[0, pl.ds(c * _L, _L)] for c in range(OUT_F // _L)]
    bps = [rsgb_v[1, pl.ds(c * _L, _L)] for c in range(OUT_F // _L)]
    idx_b = [idx_v0, idx_v1]
    wn_b = [wn_v0, wn_v1]
    rows_b = [rows_v0, rows_v1]
    y2_b = [y2_v0, y2_v1]
    gsems = [gsem0, gsem1]
    wsems = [wsem0, wsem1]

    def stage(ci):
        par = ci % 2
        q0 = base_q + ci * _CH
        pltpu.sync_copy(idxf_hbm.at[pl.ds(q0 * K, _CH * K)], idx_b[par])
        g = pltpu.async_copy(f1_hbm.at[idx_b[par]], rows_b[par], gsems[par])
        pltpu.sync_copy(wnf_hbm.at[pl.ds(q0 * K, _CH * K)],
                        wn_b[par].at[pl.ds(0, _CH * K)])
        pltpu.sync_copy(y2_hbm.at[pl.ds(q0, _CH)], y2_b[par])
        return g

    g = [None, None]
    wb = [None, None]
    g[0] = stage(0)
    for ci in range(_NCH):
        par = ci % 2
        g[par].wait()
        if ci + 1 < _NCH:
            if wb[(ci + 1) % 2] is not None:
                wb[(ci + 1) % 2].wait()
            g[(ci + 1) % 2] = stage(ci + 1)
        wnb, rowsb, y2b = wn_b[par], rows_b[par], y2_b[par]

        @plsc.parallel_loop(0, _CH, unroll=4)
        def _(qi, wnb=wnb, rowsb=rowsb, y2b=y2b):
            i3 = qi * K
            wv = wnb[pl.ds(i3, _L)]
            w0 = wv[splat0]
            w1 = wv[splat1]
            w2 = wv[splat2]
            for c in range(OUT_F // _L):
                sl = pl.ds(c * _L, _L)
                f2 = jnp.maximum(y2b[qi, sl] * rsgs[c] + bps[c], 0.0)
                acc = f2 + w0 * rowsb[i3, sl]
                acc = acc + w1 * rowsb[i3 + 1, sl]
                acc = acc + w2 * rowsb[i3 + 2, sl]
                y2b[qi, sl] = acc
        q0 = base_q + ci * _CH
        wb[par] = pltpu.async_copy(y2b, out_hbm.at[pl.ds(q0, _CH)], wsems[par])
    wb[0].wait()
    wb[1].wait()


def _sc_gather(f1, idxf, wnf, y2, rsgb):
    mesh = plsc.VectorSubcoreMesh(core_axis_name="c", subcore_axis_name="s")
    fn = pl.kernel(
        _sc_body,
        out_type=jax.ShapeDtypeStruct((N2, OUT_F), jnp.float32),
        mesh=mesh,
        scratch_types=[
            pltpu.VMEM((_CH * K,), jnp.int32),
            pltpu.VMEM((_CH * K,), jnp.int32),
            pltpu.VMEM((_CH * K + _L,), jnp.float32),
            pltpu.VMEM((_CH * K + _L,), jnp.float32),
            pltpu.VMEM((_CH * K, OUT_F), jnp.float32),
            pltpu.VMEM((_CH * K, OUT_F), jnp.float32),
            pltpu.VMEM((_CH, OUT_F), jnp.float32),
            pltpu.VMEM((_CH, OUT_F), jnp.float32),
            pltpu.VMEM((2, OUT_F), jnp.float32),
            pltpu.SemaphoreType.DMA,
            pltpu.SemaphoreType.DMA,
            pltpu.SemaphoreType.DMA,
            pltpu.SemaphoreType.DMA,
        ],
    )
    return fn(f1, idxf, wnf, y2, rsgb)


def kernel(features_1, positions_1, batch_1, features_2, positions_2, batch_2,
           W1, b1, g1, be1, W2, b2, g2, be2):
    f1, idx, wn, y2, rsgb = _tc_fused(
        features_1, W1, b1, g1, be1,
        features_2, W2, b2, g2, be2,
        positions_1, positions_2)
    out = _sc_gather(f1, idx.reshape(-1), wn.reshape(-1), y2, rsgb)
    return (out, positions_2, batch_2)


# speedup vs baseline: 1.3011x; 1.1019x over previous
import functools

import jax
import jax.numpy as jnp
from jax import lax
from jax.experimental import pallas as pl
from jax.experimental.pallas import tpu as pltpu
from jax.experimental.pallas import tpu_sc as plsc

N1 = 2048
N2 = 8192
IN_F = 512
OUT_F = 256
K = 3

QBLK = 1024
_NSTEPS = N2 // QBLK


def _tc_body(x1_ref, w1_ref, b1_ref, g1_ref, be1_ref,
             x2_ref, w2_ref, b2_ref, g2_ref, be2_ref,
             p2_ref, p1t_ref,
             f1_ref, idx_ref, wn_ref, y2_ref, rsgb_ref,
             s1_ref, s2_ref):
    i = pl.program_id(0)

    @pl.when(i == 0)
    def _():
        y = jnp.dot(x1_ref[...], w1_ref[...],
                    preferred_element_type=jnp.float32) + b1_ref[...]
        mu = jnp.sum(y, 0, keepdims=True) * (1.0 / N1)
        d = y - mu
        var = jnp.sum(d * d, 0, keepdims=True) * (1.0 / N1)
        yn = d / jnp.sqrt(var + 1e-5)
        f1_ref[...] = jnp.maximum(g1_ref[...] * yn + be1_ref[...], 0.0)

    y2 = jnp.dot(x2_ref[...], w2_ref[...],
                 preferred_element_type=jnp.float32) + b2_ref[...]
    y2_ref[...] = y2
    ps1 = jnp.sum(y2, 0, keepdims=True)
    ps2 = jnp.sum(y2 * y2, 0, keepdims=True)

    @pl.when(i == 0)
    def _():
        s1_ref[...] = ps1
        s2_ref[...] = ps2

    @pl.when(i > 0)
    def _():
        s1_ref[...] += ps1
        s2_ref[...] += ps2

    @pl.when(i == _NSTEPS - 1)
    def _():
        mu = s1_ref[...] * (1.0 / N2)
        var = s2_ref[...] * (1.0 / N2) - mu * mu
        rsg = g2_ref[...] / jnp.sqrt(var + 1e-5)
        rsgb_ref[0:1, :] = rsg
        rsgb_ref[1:2, :] = be2_ref[...] - mu * rsg

    qx = p2_ref[:, 0:1]
    qy = p2_ref[:, 1:2]
    qz = p2_ref[:, 2:3]
    dx = qx - p1t_ref[0:1, :]
    dy = qy - p1t_ref[1:2, :]
    dz = qz - p1t_ref[2:3, :]
    d2 = dx * dx + dy * dy + dz * dz
    lane = lax.broadcasted_iota(jnp.int32, (QBLK, N1), 1)
    ws = []
    for j in range(K):
        m = jnp.min(d2, axis=1, keepdims=True)
        cand = jnp.where(d2 == m, lane, N1)
        imin = jnp.min(cand, axis=1, keepdims=True)
        if j < K - 1:
            d2 = jnp.where(cand == imin, jnp.inf, d2)
        w = 1.0 / jnp.maximum(m, 1e-16)
        idx_ref[:, j : j + 1] = imin
        ws.append(w)
    den = (ws[0] + ws[1]) + ws[2]
    for j in range(K):
        wn_ref[:, j : j + 1] = ws[j] / den


def _tc_fused(x1, W1, b1, g1, be1, x2, W2, b2, g2, be2, p1, p2):
    p1t = p1.T
    r = lambda v: v.reshape(1, OUT_F)
    return pl.pallas_call(
        _tc_body,
        grid=(_NSTEPS,),
        in_specs=[
            pl.BlockSpec((N1, IN_F), lambda i: (0, 0)),
            pl.BlockSpec((IN_F, OUT_F), lambda i: (0, 0)),
            pl.BlockSpec((1, OUT_F), lambda i: (0, 0)),
            pl.BlockSpec((1, OUT_F), lambda i: (0, 0)),
            pl.BlockSpec((1, OUT_F), lambda i: (0, 0)),
            pl.BlockSpec((QBLK, IN_F), lambda i: (i, 0)),
            pl.BlockSpec((IN_F, OUT_F), lambda i: (0, 0)),
            pl.BlockSpec((1, OUT_F), lambda i: (0, 0)),
            pl.BlockSpec((1, OUT_F), lambda i: (0, 0)),
            pl.BlockSpec((1, OUT_F), lambda i: (0, 0)),
            pl.BlockSpec((QBLK, 3), lambda i: (i, 0)),
            pl.BlockSpec((3, N1), lambda i: (0, 0)),
        ],
        out_specs=[
            pl.BlockSpec((N1, OUT_F), lambda i: (0, 0)),
            pl.BlockSpec((QBLK, K), lambda i: (i, 0)),
            pl.BlockSpec((QBLK, K), lambda i: (i, 0)),
            pl.BlockSpec((QBLK, OUT_F), lambda i: (i, 0)),
            pl.BlockSpec((2, OUT_F), lambda i: (0, 0)),
        ],
        out_shape=[
            jax.ShapeDtypeStruct((N1, OUT_F), jnp.float32),
            jax.ShapeDtypeStruct((N2, K), jnp.int32),
            jax.ShapeDtypeStruct((N2, K), jnp.float32),
            jax.ShapeDtypeStruct((N2, OUT_F), jnp.float32),
            jax.ShapeDtypeStruct((2, OUT_F), jnp.float32),
        ],
        scratch_shapes=[
            pltpu.VMEM((1, OUT_F), jnp.float32),
            pltpu.VMEM((1, OUT_F), jnp.float32),
        ],
    )(x1, W1, r(b1), r(g1), r(be1), x2, W2, r(b2), r(g2), r(be2), p2, p1t)


_NC = 2
_NS = 16
_NW = _NC * _NS
_L = 16
_QW = N2 // _NW
_CH = 32
_NCH = _QW // _CH


def _sc_body(f1_hbm, idxf_hbm, wnf_hbm, y2_hbm, rsgb_hbm, out_hbm,
             idx_v0, idx_v1, wn_v0, wn_v1, rows_v0, rows_v1, y2_v0, y2_v1,
             rsgb_v, gsem0, gsem1, wsem0, wsem1, isem0, isem1, ssem0, ssem1):
    wid = lax.axis_index("s") * _NC + lax.axis_index("c")
    base_q = wid * _QW
    splat0 = jnp.zeros((_L,), jnp.int32)
    splat1 = jnp.ones((_L,), jnp.int32)
    splat2 = jnp.full((_L,), 2, jnp.int32)
    pltpu.sync_copy(rsgb_hbm, rsgb_v)
    rsgs = [rsgb_v[0, pl.ds(c * _L, _L)] for c in range(OUT_F // _L)]
    bps = [rsgb_v[1, pl.ds(c * _L, _L)] for c in range(OUT_F // _L)]
    idx_b = [idx_v0, idx_v1]
    wn_b = [wn_v0, wn_v1]
    rows_b = [rows_v0, rows_v1]
    y2_b = [y2_v0, y2_v1]
    gsems = [gsem0, gsem1]
    wsems = [wsem0, wsem1]
    isems = [isem0, isem1]
    ssems = [ssem0, ssem1]

    def copy_idx(ci):
        par = ci % 2
        q0 = base_q + ci * _CH
        return pltpu.async_copy(idxf_hbm.at[pl.ds(q0 * K, _CH * K)],
                                idx_b[par], isems[par])

    def copy_side(ci):
        par = ci % 2
        q0 = base_q + ci * _CH
        h1 = pltpu.async_copy(wnf_hbm.at[pl.ds(q0 * K, _CH * K)],
                              wn_b[par].at[pl.ds(0, _CH * K)], ssems[par])
        h2 = pltpu.async_copy(y2_hbm.at[pl.ds(q0, _CH)], y2_b[par],
                              ssems[par])
        return (h1, h2)

    def start_gather(ci):
        par = ci % 2
        return pltpu.async_copy(f1_hbm.at[idx_b[par]], rows_b[par],
                                gsems[par])

    g = [None, None]
    wb = [None, None]
    iw = [None, None]
    sw = [None, None]
    iw[0] = copy_idx(0)
    sw[0] = copy_side(0)
    if _NCH > 1:
        iw[1] = copy_idx(1)
    iw[0].wait()
    g[0] = start_gather(0)
    for ci in range(_NCH):
        par = ci % 2
        g[par].wait()
        if ci + 1 < _NCH:
            nxt = (ci + 1) % 2
            if wb[nxt] is not None:
                wb[nxt].wait()
            iw[nxt].wait()
            g[nxt] = start_gather(ci + 1)
            sw[nxt] = copy_side(ci + 1)
            if ci + 2 < _NCH:
                iw[par] = copy_idx(ci + 2)
        sw[par][0].wait()
        sw[par][1].wait()
        wnb, rowsb, y2b = wn_b[par], rows_b[par], y2_b[par]

        @plsc.parallel_loop(0, _CH, unroll=4)
        def _(qi, wnb=wnb, rowsb=rowsb, y2b=y2b):
            i3 = qi * K
            wv = wnb[pl.ds(i3, _L)]
            w0 = wv[splat0]
            w1 = wv[splat1]
            w2 = wv[splat2]
            for c in range(OUT_F // _L):
                sl = pl.ds(c * _L, _L)
                f2 = jnp.maximum(y2b[qi, sl] * rsgs[c] + bps[c], 0.0)
                acc = f2 + w0 * rowsb[i3, sl]
                acc = acc + w1 * rowsb[i3 + 1, sl]
                acc = acc + w2 * rowsb[i3 + 2, sl]
                y2b[qi, sl] = acc
        q0 = base_q + ci * _CH
        wb[par] = pltpu.async_copy(y2b, out_hbm.at[pl.ds(q0, _CH)], wsems[par])
    wb[0].wait()
    wb[1].wait()


def _sc_gather(f1, idxf, wnf, y2, rsgb):
    mesh = plsc.VectorSubcoreMesh(core_axis_name="c", subcore_axis_name="s")
    fn = pl.kernel(
        _sc_body,
        out_type=jax.ShapeDtypeStruct((N2, OUT_F), jnp.float32),
        mesh=mesh,
        scratch_types=[
            pltpu.VMEM((_CH * K,), jnp.int32),
            pltpu.VMEM((_CH * K,), jnp.int32),
            pltpu.VMEM((_CH * K + _L,), jnp.float32),
            pltpu.VMEM((_CH * K + _L,), jnp.float32),
            pltpu.VMEM((_CH * K, OUT_F), jnp.float32),
            pltpu.VMEM((_CH * K, OUT_F), jnp.float32),
            pltpu.VMEM((_CH, OUT_F), jnp.float32),
            pltpu.VMEM((_CH, OUT_F), jnp.float32),
            pltpu.VMEM((2, OUT_F), jnp.float32),
            pltpu.SemaphoreType.DMA,
            pltpu.SemaphoreType.DMA,
            pltpu.SemaphoreType.DMA,
            pltpu.SemaphoreType.DMA,
            pltpu.SemaphoreType.DMA,
            pltpu.SemaphoreType.DMA,
            pltpu.SemaphoreType.DMA,
            pltpu.SemaphoreType.DMA,
        ],
    )
    return fn(f1, idxf, wnf, y2, rsgb)


def kernel(features_1, positions_1, batch_1, features_2, positions_2, batch_2,
           W1, b1, g1, be1, W2, b2, g2, be2):
    f1, idx, wn, y2, rsgb = _tc_fused(
        features_1, W1, b1, g1, be1,
        features_2, W2, b2, g2, be2,
        positions_1, positions_2)
    out = _sc_gather(f1, idx.reshape(-1), wn.reshape(-1), y2, rsgb)
    return (out, positions_2, batch_2)
